# baseline (speedup 1.0000x reference)
import functools

import jax
import jax.numpy as jnp
from jax import lax
from jax.experimental import pallas as pl
from jax.experimental.pallas import tpu as pltpu
from jax.experimental.pallas import tpu_sc as plsc

B = 16384
EMB = 96
NS = 8
DIM = 32
FUSED = NS * NS * NS

NW = 32
BPW = B // NW
CHUNK = 64
NCH = BPW // CHUNK
L = 16


def _fuse_tables_body(tw_ref, vw_ref, lw_ref, out_ref):
    r = lax.broadcasted_iota(jnp.int32, (FUSED, DIM), 0)
    rt = r // 64
    rv = (r // 8) % NS
    rl = r % NS
    t_big = jnp.zeros((FUSED, DIM), jnp.float32)
    v_big = jnp.zeros((FUSED, DIM), jnp.float32)
    l_big = jnp.zeros((FUSED, DIM), jnp.float32)
    for s in range(NS):
        t_big = jnp.where(rt == s, tw_ref[s, :][None, :], t_big)
        v_big = jnp.where(rv == s, vw_ref[s, :][None, :], v_big)
        l_big = jnp.where(rl == s, lw_ref[s, :][None, :], l_big)
    pad = jnp.zeros((FUSED, 128 - EMB), jnp.float32)
    out_ref[...] = jnp.concatenate([t_big, v_big, l_big, pad], axis=1)


_fuse_tables = pl.pallas_call(
    _fuse_tables_body,
    out_shape=jax.ShapeDtypeStruct((FUSED, 128), jnp.float32),
)


@functools.lru_cache(maxsize=1)
def _make_sc_embed():
    mesh = plsc.VectorSubcoreMesh(core_axis_name="c", subcore_axis_name="s")

    @functools.partial(
        pl.kernel,
        out_type=jax.ShapeDtypeStruct((B, 128), jnp.float32),
        mesh=mesh,
        scratch_types=[
            pltpu.VMEM((BPW,), jnp.int32),
            pltpu.VMEM((BPW,), jnp.int32),
            pltpu.VMEM((BPW,), jnp.int32),
            pltpu.VMEM((NCH, CHUNK), jnp.int32),
            pltpu.VMEM((BPW, 128), jnp.float32),
            pltpu.SemaphoreType.DMA,
            pltpu.SemaphoreType.DMA,
            pltpu.SemaphoreType.DMA,
        ],
        compiler_params=pltpu.CompilerParams(use_tc_tiling_on_sc=True),
    )
    def _sc_embed(fused_hbm, t_hbm, v_hbm, l_hbm, out_hbm,
                  t_v, v_v, l_v, idx_v, rows_v, sem_in, sem_g, sem_w):
        wid = lax.axis_index("s") * 2 + lax.axis_index("c")
        base = wid * BPW

        hin = [
            pltpu.async_copy(t_hbm.at[pl.ds(base, BPW)], t_v, sem_in),
            pltpu.async_copy(v_hbm.at[pl.ds(base, BPW)], v_v, sem_in),
            pltpu.async_copy(l_hbm.at[pl.ds(base, BPW)], l_v, sem_in),
        ]
        for h in hin:
            h.wait()

        gh = []
        for j in range(NCH):
            for i in range(CHUNK // L):
                off = j * CHUNK + i * L
                t = jnp.clip(t_v[pl.ds(off, L)], 0, NS - 1)
                v = jnp.clip(v_v[pl.ds(off, L)], 0, NS - 1)
                l = jnp.clip(l_v[pl.ds(off, L)], 0, NS - 1)
                idx_v[j, pl.ds(i * L, L)] = t * 64 + v * 8 + l
            gh.append(
                pltpu.async_copy(
                    fused_hbm.at[idx_v.at[j]],
                    rows_v.at[pl.ds(j * CHUNK, CHUNK)],
                    sem_g,
                )
            )

        wh = []
        for j in range(NCH):
            gh[j].wait()
            wh.append(
                pltpu.async_copy(
                    rows_v.at[pl.ds(j * CHUNK, CHUNK)],
                    out_hbm.at[pl.ds(base + j * CHUNK, CHUNK)],
                    sem_w,
                )
            )
        for h in wh:
            h.wait()

    return _sc_embed


@jax.jit
def kernel(trend_state, vol_state, liq_state, trend_w, vol_w, liq_w):
    fused = _fuse_tables(trend_w, vol_w, liq_w)
    padded = _make_sc_embed()(fused, trend_state, vol_state, liq_state)
    return padded[:, :EMB]


# speedup vs baseline: 1.0033x; 1.0033x over previous
import functools

import jax
import jax.numpy as jnp
from jax import lax
from jax.experimental import pallas as pl
from jax.experimental.pallas import tpu as pltpu
from jax.experimental.pallas import tpu_sc as plsc

B = 16384
EMB = 96
NS = 8
DIM = 32
FUSED = NS * NS * NS

NW = 32
BPW = B // NW
CHUNK = 128
NCH = BPW // CHUNK
L = 16


def _fuse_tables_body(tw_ref, vw_ref, lw_ref, out_ref):
    r = lax.broadcasted_iota(jnp.int32, (FUSED, DIM), 0)
    rt = r // 64
    rv = (r // 8) % NS
    rl = r % NS
    t_big = jnp.zeros((FUSED, DIM), jnp.float32)
    v_big = jnp.zeros((FUSED, DIM), jnp.float32)
    l_big = jnp.zeros((FUSED, DIM), jnp.float32)
    for s in range(NS):
        t_big = jnp.where(rt == s, tw_ref[s, :][None, :], t_big)
        v_big = jnp.where(rv == s, vw_ref[s, :][None, :], v_big)
        l_big = jnp.where(rl == s, lw_ref[s, :][None, :], l_big)
    pad = jnp.zeros((FUSED, 128 - EMB), jnp.float32)
    out_ref[...] = jnp.concatenate([t_big, v_big, l_big, pad], axis=1)


_fuse_tables = pl.pallas_call(
    _fuse_tables_body,
    out_shape=jax.ShapeDtypeStruct((FUSED, 128), jnp.float32),
)


@functools.lru_cache(maxsize=1)
def _make_sc_embed():
    mesh = plsc.VectorSubcoreMesh(core_axis_name="c", subcore_axis_name="s")

    @functools.partial(
        pl.kernel,
        out_type=jax.ShapeDtypeStruct((B, 128), jnp.float32),
        mesh=mesh,
        scratch_types=[
            pltpu.VMEM((BPW,), jnp.int32),
            pltpu.VMEM((BPW,), jnp.int32),
            pltpu.VMEM((BPW,), jnp.int32),
            pltpu.VMEM((NCH, CHUNK), jnp.int32),
            pltpu.VMEM((BPW, 128), jnp.float32),
            pltpu.SemaphoreType.DMA,
            pltpu.SemaphoreType.DMA,
            pltpu.SemaphoreType.DMA,
        ],
        compiler_params=pltpu.CompilerParams(use_tc_tiling_on_sc=True),
    )
    def _sc_embed(fused_hbm, t_hbm, v_hbm, l_hbm, out_hbm,
                  t_v, v_v, l_v, idx_v, rows_v, sem_in, sem_g, sem_w):
        wid = lax.axis_index("s") * 2 + lax.axis_index("c")
        base = wid * BPW

        hin = [
            pltpu.async_copy(t_hbm.at[pl.ds(base, BPW)], t_v, sem_in),
            pltpu.async_copy(v_hbm.at[pl.ds(base, BPW)], v_v, sem_in),
            pltpu.async_copy(l_hbm.at[pl.ds(base, BPW)], l_v, sem_in),
        ]
        for h in hin:
            h.wait()

        gh = []
        for j in range(NCH):
            for i in range(CHUNK // L):
                off = j * CHUNK + i * L
                t = jnp.clip(t_v[pl.ds(off, L)], 0, NS - 1)
                v = jnp.clip(v_v[pl.ds(off, L)], 0, NS - 1)
                l = jnp.clip(l_v[pl.ds(off, L)], 0, NS - 1)
                idx_v[j, pl.ds(i * L, L)] = t * 64 + v * 8 + l
            gh.append(
                pltpu.async_copy(
                    fused_hbm.at[idx_v.at[j]],
                    rows_v.at[pl.ds(j * CHUNK, CHUNK)],
                    sem_g,
                )
            )

        wh = []
        for j in range(NCH):
            gh[j].wait()
            wh.append(
                pltpu.async_copy(
                    rows_v.at[pl.ds(j * CHUNK, CHUNK)],
                    out_hbm.at[pl.ds(base + j * CHUNK, CHUNK)],
                    sem_w,
                )
            )
        for h in wh:
            h.wait()

    return _sc_embed


@jax.jit
def kernel(trend_state, vol_state, liq_state, trend_w, vol_w, liq_w):
    fused = _fuse_tables(trend_w, vol_w, liq_w)
    padded = _make_sc_embed()(fused, trend_state, vol_state, liq_state)
    return jnp.concatenate([padded[:, :DIM], padded[:, DIM:EMB]], axis=1)


# speedup vs baseline: 1.0095x; 1.0062x over previous
import functools

import jax
import jax.numpy as jnp
from jax import lax
from jax.experimental import pallas as pl
from jax.experimental.pallas import tpu as pltpu
from jax.experimental.pallas import tpu_sc as plsc

B = 16384
EMB = 96
NS = 8
DIM = 32
FUSED = NS * NS * NS

NW = 32
BPW = B // NW
CHUNK = 128
NCH = BPW // CHUNK
L = 16


def _fuse_tables_body(tw_ref, vw_ref, lw_ref, out_ref):
    r = lax.broadcasted_iota(jnp.int32, (FUSED, DIM), 0)
    rt = r // 64
    rv = (r // 8) % NS
    rl = r % NS
    t_big = jnp.zeros((FUSED, DIM), jnp.float32)
    v_big = jnp.zeros((FUSED, DIM), jnp.float32)
    l_big = jnp.zeros((FUSED, DIM), jnp.float32)
    for s in range(NS):
        t_big = jnp.where(rt == s, tw_ref[s, :][None, :], t_big)
        v_big = jnp.where(rv == s, vw_ref[s, :][None, :], v_big)
        l_big = jnp.where(rl == s, lw_ref[s, :][None, :], l_big)
    pad = jnp.zeros((FUSED, 128 - EMB), jnp.float32)
    out_ref[...] = jnp.concatenate([t_big, v_big, l_big, pad], axis=1)


_fuse_tables = pl.pallas_call(
    _fuse_tables_body,
    out_shape=jax.ShapeDtypeStruct((FUSED, 128), jnp.float32),
)


@functools.lru_cache(maxsize=1)
def _make_sc_embed():
    mesh = plsc.VectorSubcoreMesh(core_axis_name="c", subcore_axis_name="s")

    @functools.partial(
        pl.kernel,
        out_type=jax.ShapeDtypeStruct((B, 128), jnp.float32),
        mesh=mesh,
        scratch_types=[
            pltpu.VMEM((BPW,), jnp.int32),
            pltpu.VMEM((BPW,), jnp.int32),
            pltpu.VMEM((BPW,), jnp.int32),
            pltpu.VMEM((NCH, CHUNK), jnp.int32),
            pltpu.VMEM((BPW, 128), jnp.float32),
            pltpu.SemaphoreType.DMA,
            pltpu.SemaphoreType.DMA,
            pltpu.SemaphoreType.DMA,
        ],
        compiler_params=pltpu.CompilerParams(use_tc_tiling_on_sc=True),
    )
    def _sc_embed(fused_hbm, t_hbm, v_hbm, l_hbm, out_hbm,
                  t_v, v_v, l_v, idx_v, rows_v, sem_in, sem_g, sem_w):
        wid = lax.axis_index("s") * 2 + lax.axis_index("c")
        base = wid * BPW

        hin = [
            pltpu.async_copy(t_hbm.at[pl.ds(base, BPW)], t_v, sem_in),
            pltpu.async_copy(v_hbm.at[pl.ds(base, BPW)], v_v, sem_in),
            pltpu.async_copy(l_hbm.at[pl.ds(base, BPW)], l_v, sem_in),
        ]
        for h in hin:
            h.wait()

        gh = []
        for j in range(NCH):
            for i in range(CHUNK // L):
                off = j * CHUNK + i * L
                t = jnp.clip(t_v[pl.ds(off, L)], 0, NS - 1)
                v = jnp.clip(v_v[pl.ds(off, L)], 0, NS - 1)
                l = jnp.clip(l_v[pl.ds(off, L)], 0, NS - 1)
                idx_v[j, pl.ds(i * L, L)] = t * 64 + v * 8 + l
            gh.append(
                pltpu.async_copy(
                    fused_hbm.at[idx_v.at[j]],
                    rows_v.at[pl.ds(j * CHUNK, CHUNK)],
                    sem_g,
                )
            )

        wh = []
        for j in range(NCH):
            gh[j].wait()
            wh.append(
                pltpu.async_copy(
                    rows_v.at[pl.ds(j * CHUNK, CHUNK)],
                    out_hbm.at[pl.ds(base + j * CHUNK, CHUNK)],
                    sem_w,
                )
            )
        for h in wh:
            h.wait()

    return _sc_embed


@jax.jit
def kernel(trend_state, vol_state, liq_state, trend_w, vol_w, liq_w):
    fused = _fuse_tables(trend_w, vol_w, liq_w)
    padded = _make_sc_embed()(fused, trend_state, vol_state, liq_state)
    return padded[:, :EMB]
